# Initial kernel scaffold; baseline (speedup 1.0000x reference)
import jax
import jax.numpy as jnp
from jax.experimental import pallas as pl


def kernel(x, edge_index, W1, b1, W2, b2):
    raise NotImplementedError("write your pallas kernel here")


# speedup vs baseline: 8.8660x; 8.8660x over previous
import functools

import jax
import jax.numpy as jnp
from jax import lax
from jax.experimental import pallas as pl
from jax.experimental.pallas import tpu as pltpu
from jax.experimental.pallas import tpu_sc as plsc

N = 50000
E = 1600000
IN_C = 256
HID_C = 128
OUT_C = 64

NPAD = 50176
SINK = NPAD - 1
CH = 128
NBUF = 4
TILES = 16
EP = 1638400
NB = 2000

_sc_mesh = functools.partial(
    plsc.VectorSubcoreMesh, core_axis_name="c", subcore_axis_name="s",
    num_cores=2, num_subcores=TILES)
_sc_params = pltpu.CompilerParams(use_tc_tiling_on_sc=False)


def _deg_body(dstp, ones_h, zeros_h, deg0, deg1, idx_v, ones_v, accum, sem):
    c = lax.axis_index("c")
    s = lax.axis_index("s")
    rpt = NPAD // TILES
    nchunk = EP // (2 * TILES * CH)
    pltpu.sync_copy(ones_h, ones_v)
    pltpu.sync_copy(zeros_h.at[pl.ds(s * rpt, rpt)], accum.at[pl.ds(s * rpt, rpt)])
    plsc.subcore_barrier()

    base0 = (c * TILES + s) * nchunk * CH

    def group(g, _):
        for b in range(NBUF):
            pltpu.sync_copy(
                dstp.at[pl.ds(base0 + (g * NBUF + b) * CH, CH)], idx_v.at[b])
        ad = [pltpu.async_copy(ones_v, accum.at[idx_v.at[b]], sem, add=True)
              for b in range(NBUF)]
        for d in ad:
            d.wait()
        return 0

    lax.fori_loop(0, nchunk // NBUF, group, 0, unroll=False)
    plsc.subcore_barrier()

    @pl.when(c == 0)
    def _():
        pltpu.sync_copy(accum.at[pl.ds(s * rpt, rpt)], deg0.at[pl.ds(s * rpt, rpt)])

    @pl.when(c == 1)
    def _():
        pltpu.sync_copy(accum.at[pl.ds(s * rpt, rpt)], deg1.at[pl.ds(s * rpt, rpt)])


def _deg_kernel(dstp, ones_h, zeros_h):
    return pl.kernel(
        _deg_body,
        out_type=[jax.ShapeDtypeStruct((NPAD, 16), jnp.float32),
                  jax.ShapeDtypeStruct((NPAD, 16), jnp.float32)],
        mesh=_sc_mesh(),
        scratch_types=[
            pltpu.VMEM((NBUF, CH), jnp.int32),
            pltpu.VMEM((CH, 16), jnp.float32),
            pltpu.VMEM_SHARED((NPAD, 16), jnp.float32),
            pltpu.SemaphoreType.DMA,
        ],
        compiler_params=_sc_params,
    )(dstp, ones_h, zeros_h)


def _make_seg_body(ngroups, cg):
    gp = ngroups // 2

    def body(*refs):
        tables = refs[:ngroups]
        srcp, dstp, zeros_h = refs[ngroups:ngroups + 3]
        outs = refs[ngroups + 3:2 * ngroups + 3]
        (src_v, dst_v, rows_v, accum, gsem, asem) = refs[2 * ngroups + 3:]
        c = lax.axis_index("c")
        s = lax.axis_index("s")
        rpt = NPAD // TILES
        nchunk = EP // (TILES * CH)
        base0 = s * nchunk * CH

        def one_pass(table, out):
            pltpu.sync_copy(zeros_h.at[pl.ds(s * rpt, rpt)],
                            accum.at[pl.ds(s * rpt, rpt)])
            plsc.subcore_barrier()

            def group(g, _):
                for b in range(NBUF):
                    k = base0 + (g * NBUF + b) * CH
                    pltpu.sync_copy(srcp.at[pl.ds(k, CH)], src_v.at[b])
                    pltpu.sync_copy(dstp.at[pl.ds(k, CH)], dst_v.at[b])
                gd = [pltpu.async_copy(table.at[src_v.at[b]], rows_v.at[b], gsem)
                      for b in range(NBUF)]
                for d in gd:
                    d.wait()
                ad = [pltpu.async_copy(rows_v.at[b], accum.at[dst_v.at[b]],
                                       asem, add=True) for b in range(NBUF)]
                for d in ad:
                    d.wait()
                return 0

            lax.fori_loop(0, nchunk // NBUF, group, 0, unroll=False)
            plsc.subcore_barrier()
            pltpu.sync_copy(accum.at[pl.ds(s * rpt, rpt)],
                            out.at[pl.ds(s * rpt, rpt)])
            plsc.subcore_barrier()

        for cc in range(2):
            @pl.when(c == cc)
            def _():
                for gi in range(gp):
                    g = cc * gp + gi
                    one_pass(tables[g], outs[g])

    return body


def _seg_sum(tables, srcp, dstp, zeros_h):
    ngroups = len(tables)
    cg = tables[0].shape[1]
    outs = pl.kernel(
        _make_seg_body(ngroups, cg),
        out_type=[jax.ShapeDtypeStruct((NPAD, cg), jnp.float32)
                  for _ in range(ngroups)],
        mesh=_sc_mesh(),
        scratch_types=[
            pltpu.VMEM((NBUF, CH), jnp.int32),
            pltpu.VMEM((NBUF, CH), jnp.int32),
            pltpu.VMEM((NBUF, CH, cg), jnp.float32),
            pltpu.VMEM_SHARED((NPAD, cg), jnp.float32),
            pltpu.SemaphoreType.DMA,
            pltpu.SemaphoreType.DMA,
        ],
        compiler_params=_sc_params,
    )(*tables, srcp, dstp, zeros_h)
    return outs


def _mm_body(x_ref, w_ref, h_ref):
    h_ref[...] = jnp.dot(x_ref[...], w_ref[...])


def _mm(x, w):
    n, k = x.shape
    m = w.shape[1]
    return pl.pallas_call(
        _mm_body,
        grid=(n // NB,),
        in_specs=[pl.BlockSpec((NB, k), lambda i: (i, 0)),
                  pl.BlockSpec((k, m), lambda i: (0, 0))],
        out_specs=pl.BlockSpec((NB, m), lambda i: (i, 0)),
        out_shape=jax.ShapeDtypeStruct((n, m), jnp.float32),
    )(x, w)


def _dinv_of(p0, p1):
    return lax.rsqrt(p0[:, :1] + p1[:, :1] + 1.0)


def _scale_body(h_ref, p0_ref, p1_ref, *out_refs):
    dinv = _dinv_of(p0_ref[...], p1_ref[...])
    s = h_ref[...] * dinv
    cg = out_refs[0].shape[1]
    for g, o in enumerate(out_refs):
        o[...] = s[:, g * cg:(g + 1) * cg]


def _scale_split(h, p0, p1, ngroups):
    n, ch = h.shape
    cg = ch // ngroups
    return pl.pallas_call(
        _scale_body,
        grid=(n // NB,),
        in_specs=[pl.BlockSpec((NB, ch), lambda i: (i, 0)),
                  pl.BlockSpec((NB, 16), lambda i: (i, 0)),
                  pl.BlockSpec((NB, 16), lambda i: (i, 0))],
        out_specs=[pl.BlockSpec((NB, cg), lambda i: (i, 0))
                   for _ in range(ngroups)],
        out_shape=[jax.ShapeDtypeStruct((n, cg), jnp.float32)
                   for _ in range(ngroups)],
    )(h, p0, p1)


def _make_combine_body(ngroups, with_mm):
    def body(*refs):
        h_ref, p0_ref, p1_ref, b_ref = refs[:4]
        accs = refs[4:4 + ngroups]
        if with_mm:
            w_ref = refs[4 + ngroups]
            out_refs = refs[5 + ngroups:]
        else:
            out_refs = refs[4 + ngroups:]
        dinv = _dinv_of(p0_ref[...], p1_ref[...])
        acc = jnp.concatenate([a[...] for a in accs], axis=1)
        h = h_ref[...]
        o = dinv * acc + (dinv * dinv) * h + b_ref[...]
        if with_mm:
            h2 = jnp.dot(o, w_ref[...])
            s2 = h2 * dinv
            out_refs[0][...] = h2
            cg = out_refs[1].shape[1]
            for g in range(len(out_refs) - 1):
                out_refs[g + 1][...] = s2[:, g * cg:(g + 1) * cg]
        else:
            out_refs[0][...] = o

    return body


def _combine(h, p0, p1, b, accs, w=None, out_groups=0):
    n, ch = h.shape
    ngroups = len(accs)
    cg = accs[0].shape[1]
    in_specs = [pl.BlockSpec((NB, ch), lambda i: (i, 0)),
                pl.BlockSpec((NB, 16), lambda i: (i, 0)),
                pl.BlockSpec((NB, 16), lambda i: (i, 0)),
                pl.BlockSpec((1, ch), lambda i: (0, 0))]
    in_specs += [pl.BlockSpec((NB, cg), lambda i: (i, 0)) for _ in accs]
    args = [h, p0, p1, b.reshape(1, ch)] + list(accs)
    if w is not None:
        m = w.shape[1]
        in_specs.append(pl.BlockSpec((ch, m), lambda i: (0, 0)))
        args.append(w)
        ocg = m // out_groups
        out_specs = [pl.BlockSpec((NB, m), lambda i: (i, 0))]
        out_specs += [pl.BlockSpec((NB, ocg), lambda i: (i, 0))
                      for _ in range(out_groups)]
        out_shape = [jax.ShapeDtypeStruct((n, m), jnp.float32)]
        out_shape += [jax.ShapeDtypeStruct((n, ocg), jnp.float32)
                      for _ in range(out_groups)]
    else:
        out_specs = [pl.BlockSpec((NB, ch), lambda i: (i, 0))]
        out_shape = [jax.ShapeDtypeStruct((n, ch), jnp.float32)]
    return pl.pallas_call(
        _make_combine_body(ngroups, w is not None),
        grid=(n // NB,),
        in_specs=in_specs,
        out_specs=out_specs,
        out_shape=out_shape,
    )(*args)


def kernel(x, edge_index, W1, b1, W2, b2):
    src = edge_index[0].astype(jnp.int32)
    dst = edge_index[1].astype(jnp.int32)
    pad = EP - E
    srcp = jnp.concatenate([src, jnp.zeros((pad,), jnp.int32)])
    dstp = jnp.concatenate([dst, jnp.full((pad,), SINK, jnp.int32)])

    ones_h = jnp.ones((CH, 16), jnp.float32)
    zeros16 = jnp.zeros((NPAD, 16), jnp.float32)
    zeros32 = jnp.zeros((NPAD, 32), jnp.float32)

    p0, p1 = _deg_kernel(dstp, ones_h, zeros16)

    h1 = _mm(x, W1)
    t = _scale_split(h1, p0[:N], p1[:N], 4)
    a1 = _seg_sum(t, srcp, dstp, zeros32)
    a1 = [a[:N] for a in a1]

    outs = _combine(h1, p0[:N], p1[:N], b1, a1, w=W2, out_groups=2)
    h2, u0, u1 = outs
    a2 = _seg_sum([u0, u1], srcp, dstp, zeros32)
    a2 = [a[:N] for a in a2]

    (z,) = _combine(h2, p0[:N], p1[:N], b2, a2)
    return z


# speedup vs baseline: 15.7466x; 1.7761x over previous
import functools

import jax
import jax.numpy as jnp
from jax import lax
from jax.experimental import pallas as pl
from jax.experimental.pallas import tpu as pltpu
from jax.experimental.pallas import tpu_sc as plsc

N = 50000
E = 1600000
IN_C = 256
HID_C = 128
OUT_C = 64

NPAD = 50176
SINK = NPAD - 1
CH = 128
NBUF = 3
TILES = 16
EP = 1634304
NB = 2000

_sc_mesh = functools.partial(
    plsc.VectorSubcoreMesh, core_axis_name="c", subcore_axis_name="s",
    num_cores=2, num_subcores=TILES)
_sc_params = pltpu.CompilerParams(use_tc_tiling_on_sc=False)


def _deg_body(dstp, ones_h, zeros_h, deg0, deg1, idx_v, ones_v, accum, sem):
    c = lax.axis_index("c")
    s = lax.axis_index("s")
    rpt = NPAD // TILES
    nchunk = EP // (2 * TILES * CH)
    pltpu.sync_copy(ones_h, ones_v)
    pltpu.sync_copy(zeros_h.at[pl.ds(s * rpt, rpt)], accum.at[pl.ds(s * rpt, rpt)])
    plsc.subcore_barrier()

    row0 = (c * TILES + s) * nchunk

    def group(g, _):
        pltpu.sync_copy(dstp.at[pl.ds(row0 + g * NBUF, NBUF)], idx_v)
        ad = [pltpu.async_copy(ones_v, accum.at[idx_v.at[b]], sem, add=True)
              for b in range(NBUF)]
        for d in ad:
            d.wait()
        return 0

    lax.fori_loop(0, nchunk // NBUF, group, 0, unroll=False)
    plsc.subcore_barrier()

    @pl.when(c == 0)
    def _():
        pltpu.sync_copy(accum.at[pl.ds(s * rpt, rpt)], deg0.at[pl.ds(s * rpt, rpt)])

    @pl.when(c == 1)
    def _():
        pltpu.sync_copy(accum.at[pl.ds(s * rpt, rpt)], deg1.at[pl.ds(s * rpt, rpt)])


def _deg_kernel(dstp, ones_h, zeros_h):
    return pl.kernel(
        _deg_body,
        out_type=[jax.ShapeDtypeStruct((NPAD, 16), jnp.float32),
                  jax.ShapeDtypeStruct((NPAD, 16), jnp.float32)],
        mesh=_sc_mesh(),
        scratch_types=[
            pltpu.VMEM((NBUF, CH), jnp.int32),
            pltpu.VMEM((CH, 16), jnp.float32),
            pltpu.VMEM_SHARED((NPAD, 16), jnp.float32),
            pltpu.SemaphoreType.DMA,
        ],
        compiler_params=_sc_params,
    )(dstp, ones_h, zeros_h)


def _make_seg_body(ngroups, cg):
    gp = ngroups // 2

    def body(*refs):
        tables = refs[:ngroups]
        srcp, dstp, zeros_h = refs[ngroups:ngroups + 3]
        outs = refs[ngroups + 3:2 * ngroups + 3]
        (src_v, dst_v, rows_v, accum, gsem, asem) = refs[2 * ngroups + 3:]
        c = lax.axis_index("c")
        s = lax.axis_index("s")
        rpt = NPAD // TILES
        nchunk = EP // (TILES * CH)
        ng = nchunk // NBUF
        row0 = s * nchunk

        def one_pass(table, out):
            pltpu.sync_copy(zeros_h.at[pl.ds(s * rpt, rpt)],
                            accum.at[pl.ds(s * rpt, rpt)])
            plsc.subcore_barrier()

            def fire_gathers(h):
                for b in range(NBUF):
                    pltpu.async_copy(table.at[src_v.at[h, b]],
                                     rows_v.at[h, b], gsem)

            def wait_gathers(h):
                for b in range(NBUF):
                    pltpu.make_async_copy(table.at[src_v.at[h, b]],
                                          rows_v.at[h, b], gsem).wait()

            def fire_adds(h):
                for b in range(NBUF):
                    pltpu.async_copy(rows_v.at[h, b],
                                     accum.at[dst_v.at[h, b]], asem, add=True)

            def wait_adds(h):
                for b in range(NBUF):
                    pltpu.make_async_copy(rows_v.at[h, b],
                                          accum.at[dst_v.at[h, b]], asem).wait()

            pltpu.sync_copy(srcp.at[pl.ds(row0, NBUF)], src_v.at[0])
            pltpu.sync_copy(dstp.at[pl.ds(row0, NBUF)], dst_v.at[0])
            fire_gathers(0)
            pltpu.sync_copy(dstp.at[pl.ds(EP // CH - NBUF, NBUF)], dst_v.at[1])
            fire_adds(1)

            def pair(gg, _):
                for h in range(2):
                    g = gg * 2 + h
                    nh = 1 - h
                    wait_adds(nh)
                    rnext = row0 + jnp.minimum(g + 1, ng - 1) * NBUF
                    pltpu.sync_copy(srcp.at[pl.ds(rnext, NBUF)], src_v.at[nh])
                    pltpu.sync_copy(dstp.at[pl.ds(rnext, NBUF)], dst_v.at[nh])
                    fire_gathers(nh)
                    wait_gathers(h)
                    fire_adds(h)
                return 0

            lax.fori_loop(0, ng // 2, pair, 0, unroll=False)
            wait_gathers(0)
            wait_adds(1)
            plsc.subcore_barrier()
            pltpu.sync_copy(accum.at[pl.ds(s * rpt, rpt)],
                            out.at[pl.ds(s * rpt, rpt)])
            plsc.subcore_barrier()

        for cc in range(2):
            @pl.when(c == cc)
            def _():
                for gi in range(gp):
                    g = cc * gp + gi
                    one_pass(tables[g], outs[g])

    return body


def _seg_sum(tables, srcp, dstp, zeros_h):
    ngroups = len(tables)
    cg = tables[0].shape[1]
    outs = pl.kernel(
        _make_seg_body(ngroups, cg),
        out_type=[jax.ShapeDtypeStruct((NPAD, cg), jnp.float32)
                  for _ in range(ngroups)],
        mesh=_sc_mesh(),
        scratch_types=[
            pltpu.VMEM((2, NBUF, CH), jnp.int32),
            pltpu.VMEM((2, NBUF, CH), jnp.int32),
            pltpu.VMEM((2, NBUF, CH, cg), jnp.float32),
            pltpu.VMEM_SHARED((NPAD, cg), jnp.float32),
            pltpu.SemaphoreType.DMA,
            pltpu.SemaphoreType.DMA,
        ],
        compiler_params=_sc_params,
    )(*tables, srcp, dstp, zeros_h)
    return outs


def _mm_body(x_ref, w_ref, h_ref):
    h_ref[...] = jnp.dot(x_ref[...], w_ref[...])


def _mm(x, w):
    n, k = x.shape
    m = w.shape[1]
    return pl.pallas_call(
        _mm_body,
        grid=(n // NB,),
        in_specs=[pl.BlockSpec((NB, k), lambda i: (i, 0)),
                  pl.BlockSpec((k, m), lambda i: (0, 0))],
        out_specs=pl.BlockSpec((NB, m), lambda i: (i, 0)),
        out_shape=jax.ShapeDtypeStruct((n, m), jnp.float32),
    )(x, w)


def _dinv_of(p0, p1):
    return lax.rsqrt(p0[:, :1] + p1[:, :1] + 1.0)


def _scale_body(h_ref, p0_ref, p1_ref, *out_refs):
    dinv = _dinv_of(p0_ref[...], p1_ref[...])
    s = h_ref[...] * dinv
    cg = out_refs[0].shape[1]
    for g, o in enumerate(out_refs):
        o[...] = s[:, g * cg:(g + 1) * cg]


def _scale_split(h, p0, p1, ngroups):
    n, ch = h.shape
    cg = ch // ngroups
    return pl.pallas_call(
        _scale_body,
        grid=(n // NB,),
        in_specs=[pl.BlockSpec((NB, ch), lambda i: (i, 0)),
                  pl.BlockSpec((NB, 16), lambda i: (i, 0)),
                  pl.BlockSpec((NB, 16), lambda i: (i, 0))],
        out_specs=[pl.BlockSpec((NB, cg), lambda i: (i, 0))
                   for _ in range(ngroups)],
        out_shape=[jax.ShapeDtypeStruct((n, cg), jnp.float32)
                   for _ in range(ngroups)],
    )(h, p0, p1)


def _make_combine_body(ngroups, with_mm):
    def body(*refs):
        h_ref, p0_ref, p1_ref, b_ref = refs[:4]
        accs = refs[4:4 + ngroups]
        if with_mm:
            w_ref = refs[4 + ngroups]
            out_refs = refs[5 + ngroups:]
        else:
            out_refs = refs[4 + ngroups:]
        dinv = _dinv_of(p0_ref[...], p1_ref[...])
        acc = jnp.concatenate([a[...] for a in accs], axis=1)
        h = h_ref[...]
        o = dinv * acc + (dinv * dinv) * h + b_ref[...]
        if with_mm:
            h2 = jnp.dot(o, w_ref[...])
            s2 = h2 * dinv
            out_refs[0][...] = h2
            cg = out_refs[1].shape[1]
            for g in range(len(out_refs) - 1):
                out_refs[g + 1][...] = s2[:, g * cg:(g + 1) * cg]
        else:
            out_refs[0][...] = o

    return body


def _combine(h, p0, p1, b, accs, w=None, out_groups=0):
    n, ch = h.shape
    ngroups = len(accs)
    cg = accs[0].shape[1]
    in_specs = [pl.BlockSpec((NB, ch), lambda i: (i, 0)),
                pl.BlockSpec((NB, 16), lambda i: (i, 0)),
                pl.BlockSpec((NB, 16), lambda i: (i, 0)),
                pl.BlockSpec((1, ch), lambda i: (0, 0))]
    in_specs += [pl.BlockSpec((NB, cg), lambda i: (i, 0)) for _ in accs]
    args = [h, p0, p1, b.reshape(1, ch)] + list(accs)
    if w is not None:
        m = w.shape[1]
        in_specs.append(pl.BlockSpec((ch, m), lambda i: (0, 0)))
        args.append(w)
        ocg = m // out_groups
        out_specs = [pl.BlockSpec((NB, m), lambda i: (i, 0))]
        out_specs += [pl.BlockSpec((NB, ocg), lambda i: (i, 0))
                      for _ in range(out_groups)]
        out_shape = [jax.ShapeDtypeStruct((n, m), jnp.float32)]
        out_shape += [jax.ShapeDtypeStruct((n, ocg), jnp.float32)
                      for _ in range(out_groups)]
    else:
        out_specs = [pl.BlockSpec((NB, ch), lambda i: (i, 0))]
        out_shape = [jax.ShapeDtypeStruct((n, ch), jnp.float32)]
    return pl.pallas_call(
        _make_combine_body(ngroups, w is not None),
        grid=(n // NB,),
        in_specs=in_specs,
        out_specs=out_specs,
        out_shape=out_shape,
    )(*args)


def kernel(x, edge_index, W1, b1, W2, b2):
    src = edge_index[0].astype(jnp.int32)
    dst = edge_index[1].astype(jnp.int32)
    pad = EP - E
    srcp = jnp.concatenate([src, jnp.zeros((pad,), jnp.int32)]).reshape(EP // CH, CH)
    dstp = jnp.concatenate([dst, jnp.full((pad,), SINK, jnp.int32)]).reshape(EP // CH, CH)

    ones_h = jnp.ones((CH, 16), jnp.float32)
    zeros16 = jnp.zeros((NPAD, 16), jnp.float32)
    zeros32 = jnp.zeros((NPAD, 32), jnp.float32)

    p0, p1 = _deg_kernel(dstp, ones_h, zeros16)

    h1 = _mm(x, W1)
    t = _scale_split(h1, p0[:N], p1[:N], 4)
    a1 = _seg_sum(t, srcp, dstp, zeros32)
    a1 = [a[:N] for a in a1]

    outs = _combine(h1, p0[:N], p1[:N], b1, a1, w=W2, out_groups=2)
    h2, u0, u1 = outs
    a2 = _seg_sum([u0, u1], srcp, dstp, zeros32)
    a2 = [a[:N] for a in a2]

    (z,) = _combine(h2, p0[:N], p1[:N], b2, a2)
    return z


# speedup vs baseline: 20.3478x; 1.2922x over previous
import functools

import jax
import jax.numpy as jnp
from jax import lax
from jax.experimental import pallas as pl
from jax.experimental.pallas import tpu as pltpu
from jax.experimental.pallas import tpu_sc as plsc

N = 50000
E = 1600000
IN_C = 256
HID_C = 128
OUT_C = 64

NPAD = 50176
SINK = NPAD - 1
CH = 128
NBUF = 3
TILES = 16
EP = 1622016
NB = 2000

_sc_mesh = functools.partial(
    plsc.VectorSubcoreMesh, core_axis_name="c", subcore_axis_name="s",
    num_cores=2, num_subcores=TILES)
_sc_params = pltpu.CompilerParams(use_tc_tiling_on_sc=False)


def _deg_body(dstp, ones_h, zeros_h, deg0, deg1, idx_v, ones_v, accum, sem):
    c = lax.axis_index("c")
    s = lax.axis_index("s")
    rpt = NPAD // TILES
    nchunk = EP // (2 * TILES * CH)
    pltpu.sync_copy(ones_h, ones_v)
    pltpu.sync_copy(zeros_h.at[pl.ds(s * rpt, rpt)], accum.at[pl.ds(s * rpt, rpt)])
    plsc.subcore_barrier()

    row0 = (c * TILES + s) * nchunk

    def group(g, _):
        pltpu.sync_copy(dstp.at[pl.ds(row0 + g * NBUF, NBUF)], idx_v)
        ad = [pltpu.async_copy(ones_v, accum.at[idx_v.at[b]], sem, add=True)
              for b in range(NBUF)]
        for d in ad:
            d.wait()
        return 0

    lax.fori_loop(0, nchunk // NBUF, group, 0, unroll=False)
    plsc.subcore_barrier()

    @pl.when(c == 0)
    def _():
        pltpu.sync_copy(accum.at[pl.ds(s * rpt, rpt)], deg0.at[pl.ds(s * rpt, rpt)])

    @pl.when(c == 1)
    def _():
        pltpu.sync_copy(accum.at[pl.ds(s * rpt, rpt)], deg1.at[pl.ds(s * rpt, rpt)])


def _deg_kernel(dstp, ones_h, zeros_h):
    return pl.kernel(
        _deg_body,
        out_type=[jax.ShapeDtypeStruct((NPAD, 16), jnp.float32),
                  jax.ShapeDtypeStruct((NPAD, 16), jnp.float32)],
        mesh=_sc_mesh(),
        scratch_types=[
            pltpu.VMEM((NBUF, CH), jnp.int32),
            pltpu.VMEM((CH, 16), jnp.float32),
            pltpu.VMEM_SHARED((NPAD, 16), jnp.float32),
            pltpu.SemaphoreType.DMA,
        ],
        compiler_params=_sc_params,
    )(dstp, ones_h, zeros_h)


def _make_seg_body(ngroups, cg):
    gp = ngroups // 2

    def body(*refs):
        tables = refs[:ngroups]
        pairs, zeros_h = refs[ngroups:ngroups + 2]
        outs = refs[ngroups + 2:2 * ngroups + 2]
        (idx_v, rows_v, accum, gsem, asem, isem0, isem1) = refs[2 * ngroups + 2:]
        isems = (isem0, isem1)
        c = lax.axis_index("c")
        s = lax.axis_index("s")
        rpt = NPAD // TILES
        nchunk = EP // (TILES * CH)
        ng = nchunk // NBUF
        row0 = s * nchunk

        def one_pass(table, out):
            pltpu.sync_copy(zeros_h.at[pl.ds(s * rpt, rpt)],
                            accum.at[pl.ds(s * rpt, rpt)])
            plsc.subcore_barrier()

            def load_idx(slot, grp, sem):
                pltpu.async_copy(pairs.at[pl.ds(row0 + grp * NBUF, NBUF)],
                                 idx_v.at[slot], sem)

            def wait_idx(sem):
                pltpu.make_async_copy(pairs.at[pl.ds(row0, NBUF)],
                                      idx_v.at[0], sem).wait()

            def gathers(h, slot):
                for b in range(NBUF):
                    pltpu.async_copy(table.at[idx_v.at[slot, b, 0]],
                                     rows_v.at[h, b], gsem)

            def wait_gathers(h, slot):
                for b in range(NBUF):
                    pltpu.make_async_copy(table.at[idx_v.at[slot, b, 0]],
                                          rows_v.at[h, b], gsem).wait()

            def adds(h, slot):
                for b in range(NBUF):
                    pltpu.async_copy(rows_v.at[h, b],
                                     accum.at[idx_v.at[slot, b, 1]], asem,
                                     add=True)

            def wait_adds(h, slot):
                for b in range(NBUF):
                    pltpu.make_async_copy(rows_v.at[h, b],
                                          accum.at[idx_v.at[slot, b, 1]],
                                          asem).wait()

            pltpu.sync_copy(pairs.at[pl.ds(row0, NBUF)], idx_v.at[0])
            gathers(0, 0)
            load_idx(1, 1, isems[1])
            pltpu.sync_copy(pairs.at[pl.ds(EP // CH - NBUF, NBUF)], idx_v.at[3])
            adds(1, 3)

            def quad(gg, _):
                for q in range(4):
                    g = gg * 4 + q
                    h, nh = q % 2, 1 - q % 2
                    load_idx((q + 2) % 4, jnp.minimum(g + 2, ng - 1), isems[h])
                    wait_adds(nh, (q + 3) % 4)
                    wait_idx(isems[nh])
                    gathers(nh, (q + 1) % 4)
                    wait_gathers(h, q)
                    adds(h, q)
                return 0

            lax.fori_loop(0, ng // 4, quad, 0, unroll=False)
            wait_gathers(0, 0)
            wait_adds(1, 3)
            wait_idx(isems[1])
            plsc.subcore_barrier()
            pltpu.sync_copy(accum.at[pl.ds(s * rpt, rpt)],
                            out.at[pl.ds(s * rpt, rpt)])
            plsc.subcore_barrier()

        for cc in range(2):
            @pl.when(c == cc)
            def _():
                for gi in range(gp):
                    g = cc * gp + gi
                    one_pass(tables[g], outs[g])

    return body


def _seg_sum(tables, pairs, zeros_h):
    ngroups = len(tables)
    cg = tables[0].shape[1]
    outs = pl.kernel(
        _make_seg_body(ngroups, cg),
        out_type=[jax.ShapeDtypeStruct((NPAD, cg), jnp.float32)
                  for _ in range(ngroups)],
        mesh=_sc_mesh(),
        scratch_types=[
            pltpu.VMEM((4, NBUF, 2, CH), jnp.int32),
            pltpu.VMEM((2, NBUF, CH, cg), jnp.float32),
            pltpu.VMEM_SHARED((NPAD, cg), jnp.float32),
            pltpu.SemaphoreType.DMA,
            pltpu.SemaphoreType.DMA,
            pltpu.SemaphoreType.DMA,
            pltpu.SemaphoreType.DMA,
        ],
        compiler_params=_sc_params,
    )(*tables, pairs, zeros_h)
    return outs


def _mm_body(x_ref, w_ref, h_ref):
    h_ref[...] = jnp.dot(x_ref[...], w_ref[...])


def _mm(x, w):
    n, k = x.shape
    m = w.shape[1]
    return pl.pallas_call(
        _mm_body,
        grid=(n // NB,),
        in_specs=[pl.BlockSpec((NB, k), lambda i: (i, 0)),
                  pl.BlockSpec((k, m), lambda i: (0, 0))],
        out_specs=pl.BlockSpec((NB, m), lambda i: (i, 0)),
        out_shape=jax.ShapeDtypeStruct((n, m), jnp.float32),
    )(x, w)


def _dinv_of(p0, p1):
    return lax.rsqrt(p0[:, :1] + p1[:, :1] + 1.0)


def _scale_body(h_ref, p0_ref, p1_ref, *out_refs):
    dinv = _dinv_of(p0_ref[...], p1_ref[...])
    s = h_ref[...] * dinv
    cg = out_refs[0].shape[1]
    for g, o in enumerate(out_refs):
        o[...] = s[:, g * cg:(g + 1) * cg]


def _scale_split(h, p0, p1, ngroups):
    n, ch = h.shape
    cg = ch // ngroups
    return pl.pallas_call(
        _scale_body,
        grid=(n // NB,),
        in_specs=[pl.BlockSpec((NB, ch), lambda i: (i, 0)),
                  pl.BlockSpec((NB, 16), lambda i: (i, 0)),
                  pl.BlockSpec((NB, 16), lambda i: (i, 0))],
        out_specs=[pl.BlockSpec((NB, cg), lambda i: (i, 0))
                   for _ in range(ngroups)],
        out_shape=[jax.ShapeDtypeStruct((n, cg), jnp.float32)
                   for _ in range(ngroups)],
    )(h, p0, p1)


def _make_combine_body(ngroups, with_mm):
    def body(*refs):
        h_ref, p0_ref, p1_ref, b_ref = refs[:4]
        accs = refs[4:4 + ngroups]
        if with_mm:
            w_ref = refs[4 + ngroups]
            out_refs = refs[5 + ngroups:]
        else:
            out_refs = refs[4 + ngroups:]
        dinv = _dinv_of(p0_ref[...], p1_ref[...])
        acc = jnp.concatenate([a[...] for a in accs], axis=1)
        h = h_ref[...]
        o = dinv * acc + (dinv * dinv) * h + b_ref[...]
        if with_mm:
            h2 = jnp.dot(o, w_ref[...])
            s2 = h2 * dinv
            out_refs[0][...] = h2
            cg = out_refs[1].shape[1]
            for g in range(len(out_refs) - 1):
                out_refs[g + 1][...] = s2[:, g * cg:(g + 1) * cg]
        else:
            out_refs[0][...] = o

    return body


def _combine(h, p0, p1, b, accs, w=None, out_groups=0):
    n, ch = h.shape
    ngroups = len(accs)
    cg = accs[0].shape[1]
    in_specs = [pl.BlockSpec((NB, ch), lambda i: (i, 0)),
                pl.BlockSpec((NB, 16), lambda i: (i, 0)),
                pl.BlockSpec((NB, 16), lambda i: (i, 0)),
                pl.BlockSpec((1, ch), lambda i: (0, 0))]
    in_specs += [pl.BlockSpec((NB, cg), lambda i: (i, 0)) for _ in accs]
    args = [h, p0, p1, b.reshape(1, ch)] + list(accs)
    if w is not None:
        m = w.shape[1]
        in_specs.append(pl.BlockSpec((ch, m), lambda i: (0, 0)))
        args.append(w)
        ocg = m // out_groups
        out_specs = [pl.BlockSpec((NB, m), lambda i: (i, 0))]
        out_specs += [pl.BlockSpec((NB, ocg), lambda i: (i, 0))
                      for _ in range(out_groups)]
        out_shape = [jax.ShapeDtypeStruct((n, m), jnp.float32)]
        out_shape += [jax.ShapeDtypeStruct((n, ocg), jnp.float32)
                      for _ in range(out_groups)]
    else:
        out_specs = [pl.BlockSpec((NB, ch), lambda i: (i, 0))]
        out_shape = [jax.ShapeDtypeStruct((n, ch), jnp.float32)]
    return pl.pallas_call(
        _make_combine_body(ngroups, w is not None),
        grid=(n // NB,),
        in_specs=in_specs,
        out_specs=out_specs,
        out_shape=out_shape,
    )(*args)


def kernel(x, edge_index, W1, b1, W2, b2):
    src = edge_index[0].astype(jnp.int32)
    dst = edge_index[1].astype(jnp.int32)
    pad = EP - E
    srcp = jnp.concatenate([src, jnp.zeros((pad,), jnp.int32)]).reshape(EP // CH, CH)
    dstp = jnp.concatenate([dst, jnp.full((pad,), SINK, jnp.int32)]).reshape(EP // CH, CH)
    pairs = jnp.stack([srcp, dstp], axis=1)

    ones_h = jnp.ones((CH, 16), jnp.float32)
    zeros16 = jnp.zeros((NPAD, 16), jnp.float32)
    zeros32 = jnp.zeros((NPAD, 32), jnp.float32)

    p0, p1 = _deg_kernel(dstp, ones_h, zeros16)

    h1 = _mm(x, W1)
    t = _scale_split(h1, p0[:N], p1[:N], 4)
    a1 = _seg_sum(t, pairs, zeros32)
    a1 = [a[:N] for a in a1]

    outs = _combine(h1, p0[:N], p1[:N], b1, a1, w=W2, out_groups=2)
    h2, u0, u1 = outs
    a2 = _seg_sum([u0, u1], pairs, zeros32)
    a2 = [a[:N] for a in a2]

    (z,) = _combine(h2, p0[:N], p1[:N], b2, a2)
    return z


# speedup vs baseline: 20.5268x; 1.0088x over previous
import functools

import jax
import jax.numpy as jnp
from jax import lax
from jax.experimental import pallas as pl
from jax.experimental.pallas import tpu as pltpu
from jax.experimental.pallas import tpu_sc as plsc

N = 50000
E = 1600000
IN_C = 256
HID_C = 128
OUT_C = 64

NPAD = 50176
SINK = NPAD - 1
CH = 128
NBUF = 3
TILES = 16
EP = 1622016
NB = 2000

_sc_mesh = functools.partial(
    plsc.VectorSubcoreMesh, core_axis_name="c", subcore_axis_name="s",
    num_cores=2, num_subcores=TILES)
_sc_params = pltpu.CompilerParams(use_tc_tiling_on_sc=False)


def _deg_body(dstp, ones_h, zeros_h, deg0, deg1, idx_v, ones_v, accum, sem):
    c = lax.axis_index("c")
    s = lax.axis_index("s")
    rpt = NPAD // TILES
    nchunk = EP // (2 * TILES * CH)
    pltpu.sync_copy(ones_h, ones_v)
    pltpu.sync_copy(zeros_h.at[pl.ds(s * rpt, rpt)], accum.at[pl.ds(s * rpt, rpt)])
    plsc.subcore_barrier()

    row0 = (c * TILES + s) * nchunk

    def group(g, _):
        pltpu.sync_copy(dstp.at[pl.ds(row0 + g * NBUF, NBUF)], idx_v)
        ad = [pltpu.async_copy(ones_v, accum.at[idx_v.at[b]], sem, add=True)
              for b in range(NBUF)]
        for d in ad:
            d.wait()
        return 0

    lax.fori_loop(0, nchunk // NBUF, group, 0, unroll=False)
    plsc.subcore_barrier()

    @pl.when(c == 0)
    def _():
        pltpu.sync_copy(accum.at[pl.ds(s * rpt, rpt)], deg0.at[pl.ds(s * rpt, rpt)])

    @pl.when(c == 1)
    def _():
        pltpu.sync_copy(accum.at[pl.ds(s * rpt, rpt)], deg1.at[pl.ds(s * rpt, rpt)])


def _deg_kernel(dstp, ones_h, zeros_h):
    return pl.kernel(
        _deg_body,
        out_type=[jax.ShapeDtypeStruct((NPAD, 16), jnp.float32),
                  jax.ShapeDtypeStruct((NPAD, 16), jnp.float32)],
        mesh=_sc_mesh(),
        scratch_types=[
            pltpu.VMEM((NBUF, CH), jnp.int32),
            pltpu.VMEM((CH, 16), jnp.float32),
            pltpu.VMEM_SHARED((NPAD, 16), jnp.float32),
            pltpu.SemaphoreType.DMA,
        ],
        compiler_params=_sc_params,
    )(dstp, ones_h, zeros_h)


def _make_seg_body(ngroups, cg):
    gp = ngroups // 2

    def body(*refs):
        tables = refs[:ngroups]
        pairs, zeros_h = refs[ngroups:ngroups + 2]
        outs = refs[ngroups + 2:2 * ngroups + 2]
        (idx_v, rows_v, accum, gsem, asem, isem0, isem1) = refs[2 * ngroups + 2:]
        isems = (isem0, isem1)
        c = lax.axis_index("c")
        s = lax.axis_index("s")
        rpt = NPAD // TILES
        nchunk = EP // (TILES * CH)
        ng = nchunk // NBUF
        row0 = s * nchunk

        def one_pass(table, out):
            pltpu.sync_copy(zeros_h.at[pl.ds(s * rpt, rpt)],
                            accum.at[pl.ds(s * rpt, rpt)])
            plsc.subcore_barrier()

            def load_idx(slot, grp, sem):
                pltpu.async_copy(pairs.at[pl.ds(row0 + grp * NBUF, NBUF)],
                                 idx_v.at[slot], sem)

            def wait_idx(sem):
                pltpu.make_async_copy(pairs.at[pl.ds(row0, NBUF)],
                                      idx_v.at[0], sem).wait()

            def gathers(h, slot):
                for b in range(NBUF):
                    pltpu.async_copy(table.at[idx_v.at[slot, b, 0]],
                                     rows_v.at[h, b], gsem)

            def wait_gathers(h, slot):
                for b in range(NBUF):
                    pltpu.make_async_copy(table.at[idx_v.at[slot, b, 0]],
                                          rows_v.at[h, b], gsem).wait()

            def adds(h, slot):
                for b in range(NBUF):
                    pltpu.async_copy(rows_v.at[h, b],
                                     accum.at[idx_v.at[slot, b, 1]], asem,
                                     add=True)

            def wait_adds(h, slot):
                for b in range(NBUF):
                    pltpu.make_async_copy(rows_v.at[h, b],
                                          accum.at[idx_v.at[slot, b, 1]],
                                          asem).wait()

            pltpu.sync_copy(pairs.at[pl.ds(row0, NBUF)], idx_v.at[0])
            gathers(0, 0)
            load_idx(1, 1, isems[1])
            pltpu.sync_copy(pairs.at[pl.ds(EP // CH - NBUF, NBUF)], idx_v.at[3])
            adds(1, 3)

            def quad(gg, _):
                for q in range(4):
                    g = gg * 4 + q
                    h, nh = q % 2, 1 - q % 2
                    load_idx((q + 2) % 4, jnp.minimum(g + 2, ng - 1), isems[h])
                    wait_adds(nh, (q + 3) % 4)
                    wait_idx(isems[nh])
                    gathers(nh, (q + 1) % 4)
                    wait_gathers(h, q)
                    adds(h, q)
                return 0

            lax.fori_loop(0, ng // 4, quad, 0, unroll=False)
            wait_gathers(0, 0)
            wait_adds(1, 3)
            wait_idx(isems[1])
            plsc.subcore_barrier()
            pltpu.sync_copy(accum.at[pl.ds(s * rpt, rpt)],
                            out.at[pl.ds(s * rpt, rpt)])
            plsc.subcore_barrier()

        for cc in range(2):
            @pl.when(c == cc)
            def _():
                for gi in range(gp):
                    g = cc * gp + gi
                    one_pass(tables[g], outs[g])

    return body


def _seg_sum(tables, pairs, zeros_h):
    ngroups = len(tables)
    cg = tables[0].shape[1]
    outs = pl.kernel(
        _make_seg_body(ngroups, cg),
        out_type=[jax.ShapeDtypeStruct((NPAD, cg), jnp.float32)
                  for _ in range(ngroups)],
        mesh=_sc_mesh(),
        scratch_types=[
            pltpu.VMEM((4, NBUF, 2, CH), jnp.int32),
            pltpu.VMEM((2, NBUF, CH, cg), jnp.float32),
            pltpu.VMEM_SHARED((NPAD, cg), jnp.float32),
            pltpu.SemaphoreType.DMA,
            pltpu.SemaphoreType.DMA,
            pltpu.SemaphoreType.DMA,
            pltpu.SemaphoreType.DMA,
        ],
        compiler_params=_sc_params,
    )(*tables, pairs, zeros_h)
    return outs


def _dinv_of(p0, p1):
    return lax.rsqrt(p0[:, :1] + p1[:, :1] + 1.0)


def _mm_scale_body(x_ref, w_ref, p0_ref, p1_ref, h_ref, *out_refs):
    h = jnp.dot(x_ref[...], w_ref[...])
    dinv = _dinv_of(p0_ref[...], p1_ref[...])
    s = h * dinv
    h_ref[...] = h
    cg = out_refs[0].shape[1]
    for g, o in enumerate(out_refs):
        o[...] = s[:, g * cg:(g + 1) * cg]


def _mm_scale_split(x, w, p0, p1, ngroups):
    n, k = x.shape
    m = w.shape[1]
    cg = m // ngroups
    return pl.pallas_call(
        _mm_scale_body,
        grid=(n // NB,),
        in_specs=[pl.BlockSpec((NB, k), lambda i: (i, 0)),
                  pl.BlockSpec((k, m), lambda i: (0, 0)),
                  pl.BlockSpec((NB, 16), lambda i: (i, 0)),
                  pl.BlockSpec((NB, 16), lambda i: (i, 0))],
        out_specs=[pl.BlockSpec((NB, m), lambda i: (i, 0))]
                  + [pl.BlockSpec((NB, cg), lambda i: (i, 0))
                     for _ in range(ngroups)],
        out_shape=[jax.ShapeDtypeStruct((n, m), jnp.float32)]
                  + [jax.ShapeDtypeStruct((n, cg), jnp.float32)
                     for _ in range(ngroups)],
    )(x, w, p0, p1)


def _make_combine_body(ngroups, with_mm):
    def body(*refs):
        h_ref, p0_ref, p1_ref, b_ref = refs[:4]
        accs = refs[4:4 + ngroups]
        if with_mm:
            w_ref = refs[4 + ngroups]
            out_refs = refs[5 + ngroups:]
        else:
            out_refs = refs[4 + ngroups:]
        dinv = _dinv_of(p0_ref[...], p1_ref[...])
        acc = jnp.concatenate([a[...] for a in accs], axis=1)
        h = h_ref[...]
        o = dinv * acc + (dinv * dinv) * h + b_ref[...]
        if with_mm:
            h2 = jnp.dot(o, w_ref[...])
            s2 = h2 * dinv
            out_refs[0][...] = h2
            cg = out_refs[1].shape[1]
            for g in range(len(out_refs) - 1):
                out_refs[g + 1][...] = s2[:, g * cg:(g + 1) * cg]
        else:
            out_refs[0][...] = o

    return body


def _combine(h, p0, p1, b, accs, w=None, out_groups=0):
    n, ch = h.shape
    ngroups = len(accs)
    cg = accs[0].shape[1]
    in_specs = [pl.BlockSpec((NB, ch), lambda i: (i, 0)),
                pl.BlockSpec((NB, 16), lambda i: (i, 0)),
                pl.BlockSpec((NB, 16), lambda i: (i, 0)),
                pl.BlockSpec((1, ch), lambda i: (0, 0))]
    in_specs += [pl.BlockSpec((NB, cg), lambda i: (i, 0)) for _ in accs]
    args = [h, p0, p1, b.reshape(1, ch)] + list(accs)
    if w is not None:
        m = w.shape[1]
        in_specs.append(pl.BlockSpec((ch, m), lambda i: (0, 0)))
        args.append(w)
        ocg = m // out_groups
        out_specs = [pl.BlockSpec((NB, m), lambda i: (i, 0))]
        out_specs += [pl.BlockSpec((NB, ocg), lambda i: (i, 0))
                      for _ in range(out_groups)]
        out_shape = [jax.ShapeDtypeStruct((n, m), jnp.float32)]
        out_shape += [jax.ShapeDtypeStruct((n, ocg), jnp.float32)
                      for _ in range(out_groups)]
    else:
        out_specs = [pl.BlockSpec((NB, ch), lambda i: (i, 0))]
        out_shape = [jax.ShapeDtypeStruct((n, ch), jnp.float32)]
    return pl.pallas_call(
        _make_combine_body(ngroups, w is not None),
        grid=(n // NB,),
        in_specs=in_specs,
        out_specs=out_specs,
        out_shape=out_shape,
    )(*args)


def kernel(x, edge_index, W1, b1, W2, b2):
    src = edge_index[0].astype(jnp.int32)
    dst = edge_index[1].astype(jnp.int32)
    pad = EP - E
    srcp = jnp.concatenate([src, jnp.zeros((pad,), jnp.int32)]).reshape(EP // CH, CH)
    dstp = jnp.concatenate([dst, jnp.full((pad,), SINK, jnp.int32)]).reshape(EP // CH, CH)
    pairs = jnp.stack([srcp, dstp], axis=1)

    ones_h = jnp.ones((CH, 16), jnp.float32)
    zeros16 = jnp.zeros((NPAD, 16), jnp.float32)
    zeros32 = jnp.zeros((NPAD, 32), jnp.float32)

    p0, p1 = _deg_kernel(dstp, ones_h, zeros16)

    h1, *t = _mm_scale_split(x, W1, p0[:N], p1[:N], 4)
    a1 = _seg_sum(t, pairs, zeros32)
    a1 = [a[:N] for a in a1]

    outs = _combine(h1, p0[:N], p1[:N], b1, a1, w=W2, out_groups=2)
    h2, u0, u1 = outs
    a2 = _seg_sum([u0, u1], pairs, zeros32)
    a2 = [a[:N] for a in a2]

    (z,) = _combine(h2, p0[:N], p1[:N], b2, a2)
    return z


# speedup vs baseline: 22.1212x; 1.0777x over previous
import functools

import jax
import jax.numpy as jnp
from jax import lax
from jax.experimental import pallas as pl
from jax.experimental.pallas import tpu as pltpu
from jax.experimental.pallas import tpu_sc as plsc

N = 50000
E = 1600000
IN_C = 256
HID_C = 128
OUT_C = 64

NPAD = 50176
SINK = NPAD - 1
CH = 128
NBUF = 3
TILES = 16
EP = 1622016
NB = 2000

_sc_mesh = functools.partial(
    plsc.VectorSubcoreMesh, core_axis_name="c", subcore_axis_name="s",
    num_cores=2, num_subcores=TILES)
_sc_params = pltpu.CompilerParams(use_tc_tiling_on_sc=False)


def _deg_body(dstp, ones_h, zeros_h, deg0, deg1, idx_v, ones_v, accum, sem):
    c = lax.axis_index("c")
    s = lax.axis_index("s")
    rpt = NPAD // TILES
    nchunk = EP // (2 * TILES * CH)
    pltpu.sync_copy(ones_h, ones_v)
    pltpu.sync_copy(zeros_h.at[pl.ds(s * rpt, rpt)], accum.at[pl.ds(s * rpt, rpt)])
    plsc.subcore_barrier()

    row0 = (c * TILES + s) * nchunk

    def group(g, _):
        pltpu.sync_copy(dstp.at[pl.ds(row0 + g * NBUF, NBUF)], idx_v)
        ad = [pltpu.async_copy(ones_v, accum.at[idx_v.at[b]], sem, add=True)
              for b in range(NBUF)]
        for d in ad:
            d.wait()
        return 0

    lax.fori_loop(0, nchunk // NBUF, group, 0, unroll=False)
    plsc.subcore_barrier()

    @pl.when(c == 0)
    def _():
        pltpu.sync_copy(accum.at[pl.ds(s * rpt, rpt)], deg0.at[pl.ds(s * rpt, rpt)])

    @pl.when(c == 1)
    def _():
        pltpu.sync_copy(accum.at[pl.ds(s * rpt, rpt)], deg1.at[pl.ds(s * rpt, rpt)])


def _deg_kernel(dstp, ones_h, zeros_h):
    return pl.kernel(
        _deg_body,
        out_type=[jax.ShapeDtypeStruct((NPAD, 16), jnp.float32),
                  jax.ShapeDtypeStruct((NPAD, 16), jnp.float32)],
        mesh=_sc_mesh(),
        scratch_types=[
            pltpu.VMEM((NBUF, CH), jnp.int32),
            pltpu.VMEM((CH, 16), jnp.float32),
            pltpu.VMEM_SHARED((NPAD, 16), jnp.float32),
            pltpu.SemaphoreType.DMA,
        ],
        compiler_params=_sc_params,
    )(dstp, ones_h, zeros_h)


def _make_seg_body(ngroups, cg):
    gp = ngroups // 2

    def body(*refs):
        tables = refs[:ngroups]
        srcp, dstp, zeros_h = refs[ngroups:ngroups + 3]
        outs = refs[ngroups + 3:2 * ngroups + 3]
        (idx_v, rows_v, accum, gsem, asem, isem0, isem1) = refs[2 * ngroups + 3:]
        isems = (isem0, isem1)
        c = lax.axis_index("c")
        s = lax.axis_index("s")
        rpt = NPAD // TILES
        nchunk = EP // (TILES * CH)
        ng = nchunk // NBUF
        row0 = s * nchunk

        def one_pass(table, out):
            pltpu.sync_copy(zeros_h.at[pl.ds(s * rpt, rpt)],
                            accum.at[pl.ds(s * rpt, rpt)])
            plsc.subcore_barrier()

            def load_idx(slot, grp, sem):
                pltpu.async_copy(srcp.at[pl.ds(row0 + grp * NBUF, NBUF)],
                                 idx_v.at[slot, 0], sem)
                pltpu.async_copy(dstp.at[pl.ds(row0 + grp * NBUF, NBUF)],
                                 idx_v.at[slot, 1], sem)

            def wait_idx(sem):
                for _ in range(2):
                    pltpu.make_async_copy(srcp.at[pl.ds(row0, NBUF)],
                                          idx_v.at[0, 0], sem).wait()

            def gathers(h, slot):
                for b in range(NBUF):
                    pltpu.async_copy(table.at[idx_v.at[slot, 0, b]],
                                     rows_v.at[h, b], gsem)

            def wait_gathers(h, slot):
                for b in range(NBUF):
                    pltpu.make_async_copy(table.at[idx_v.at[slot, 0, b]],
                                          rows_v.at[h, b], gsem).wait()

            def adds(h, slot):
                for b in range(NBUF):
                    pltpu.async_copy(rows_v.at[h, b],
                                     accum.at[idx_v.at[slot, 1, b]], asem,
                                     add=True)

            def wait_adds(h, slot):
                for b in range(NBUF):
                    pltpu.make_async_copy(rows_v.at[h, b],
                                          accum.at[idx_v.at[slot, 1, b]],
                                          asem).wait()

            pltpu.sync_copy(srcp.at[pl.ds(row0, NBUF)], idx_v.at[0, 0])
            pltpu.sync_copy(dstp.at[pl.ds(row0, NBUF)], idx_v.at[0, 1])
            gathers(0, 0)
            load_idx(1, 1, isems[1])
            pltpu.sync_copy(dstp.at[pl.ds(EP // CH - NBUF, NBUF)], idx_v.at[3, 1])
            adds(1, 3)

            def quad(gg, _):
                for q in range(4):
                    g = gg * 4 + q
                    h, nh = q % 2, 1 - q % 2
                    load_idx((q + 2) % 4, jnp.minimum(g + 2, ng - 1), isems[h])
                    wait_adds(nh, (q + 3) % 4)
                    wait_idx(isems[nh])
                    gathers(nh, (q + 1) % 4)
                    wait_gathers(h, q)
                    adds(h, q)
                return 0

            lax.fori_loop(0, ng // 4, quad, 0, unroll=False)
            wait_gathers(0, 0)
            wait_adds(1, 3)
            wait_idx(isems[1])
            plsc.subcore_barrier()
            pltpu.sync_copy(accum.at[pl.ds(s * rpt, rpt)],
                            out.at[pl.ds(s * rpt, rpt)])
            plsc.subcore_barrier()

        for cc in range(2):
            @pl.when(c == cc)
            def _():
                for gi in range(gp):
                    g = cc * gp + gi
                    one_pass(tables[g], outs[g])

    return body


def _seg_sum(tables, srcp, dstp, zeros_h):
    ngroups = len(tables)
    cg = tables[0].shape[1]
    outs = pl.kernel(
        _make_seg_body(ngroups, cg),
        out_type=[jax.ShapeDtypeStruct((NPAD, cg), jnp.float32)
                  for _ in range(ngroups)],
        mesh=_sc_mesh(),
        scratch_types=[
            pltpu.VMEM((4, 2, NBUF, CH), jnp.int32),
            pltpu.VMEM((2, NBUF, CH, cg), jnp.float32),
            pltpu.VMEM_SHARED((NPAD, cg), jnp.float32),
            pltpu.SemaphoreType.DMA,
            pltpu.SemaphoreType.DMA,
            pltpu.SemaphoreType.DMA,
            pltpu.SemaphoreType.DMA,
        ],
        compiler_params=_sc_params,
    )(*tables, srcp, dstp, zeros_h)
    return outs


def _dinv_of(p0, p1):
    return lax.rsqrt(p0[:, :1] + p1[:, :1] + 1.0)


def _mm_scale_body(x_ref, w_ref, p0_ref, p1_ref, h_ref, *out_refs):
    h = jnp.dot(x_ref[...], w_ref[...])
    dinv = _dinv_of(p0_ref[...], p1_ref[...])
    s = h * dinv
    h_ref[...] = h
    cg = out_refs[0].shape[1]
    for g, o in enumerate(out_refs):
        o[...] = s[:, g * cg:(g + 1) * cg]


def _mm_scale_split(x, w, p0, p1, ngroups):
    n, k = x.shape
    m = w.shape[1]
    cg = m // ngroups
    return pl.pallas_call(
        _mm_scale_body,
        grid=(n // NB,),
        in_specs=[pl.BlockSpec((NB, k), lambda i: (i, 0)),
                  pl.BlockSpec((k, m), lambda i: (0, 0)),
                  pl.BlockSpec((NB, 16), lambda i: (i, 0)),
                  pl.BlockSpec((NB, 16), lambda i: (i, 0))],
        out_specs=[pl.BlockSpec((NB, m), lambda i: (i, 0))]
                  + [pl.BlockSpec((NB, cg), lambda i: (i, 0))
                     for _ in range(ngroups)],
        out_shape=[jax.ShapeDtypeStruct((n, m), jnp.float32)]
                  + [jax.ShapeDtypeStruct((n, cg), jnp.float32)
                     for _ in range(ngroups)],
    )(x, w, p0, p1)


def _make_combine_body(ngroups, with_mm):
    def body(*refs):
        h_ref, p0_ref, p1_ref, b_ref = refs[:4]
        accs = refs[4:4 + ngroups]
        if with_mm:
            w_ref = refs[4 + ngroups]
            out_refs = refs[5 + ngroups:]
        else:
            out_refs = refs[4 + ngroups:]
        dinv = _dinv_of(p0_ref[...], p1_ref[...])
        acc = jnp.concatenate([a[...] for a in accs], axis=1)
        h = h_ref[...]
        o = dinv * acc + (dinv * dinv) * h + b_ref[...]
        if with_mm:
            h2 = jnp.dot(o, w_ref[...])
            s2 = h2 * dinv
            out_refs[0][...] = h2
            cg = out_refs[1].shape[1]
            for g in range(len(out_refs) - 1):
                out_refs[g + 1][...] = s2[:, g * cg:(g + 1) * cg]
        else:
            out_refs[0][...] = o

    return body


def _combine(h, p0, p1, b, accs, w=None, out_groups=0):
    n, ch = h.shape
    ngroups = len(accs)
    cg = accs[0].shape[1]
    in_specs = [pl.BlockSpec((NB, ch), lambda i: (i, 0)),
                pl.BlockSpec((NB, 16), lambda i: (i, 0)),
                pl.BlockSpec((NB, 16), lambda i: (i, 0)),
                pl.BlockSpec((1, ch), lambda i: (0, 0))]
    in_specs += [pl.BlockSpec((NB, cg), lambda i: (i, 0)) for _ in accs]
    args = [h, p0, p1, b.reshape(1, ch)] + list(accs)
    if w is not None:
        m = w.shape[1]
        in_specs.append(pl.BlockSpec((ch, m), lambda i: (0, 0)))
        args.append(w)
        ocg = m // out_groups
        out_specs = [pl.BlockSpec((NB, m), lambda i: (i, 0))]
        out_specs += [pl.BlockSpec((NB, ocg), lambda i: (i, 0))
                      for _ in range(out_groups)]
        out_shape = [jax.ShapeDtypeStruct((n, m), jnp.float32)]
        out_shape += [jax.ShapeDtypeStruct((n, ocg), jnp.float32)
                      for _ in range(out_groups)]
    else:
        out_specs = [pl.BlockSpec((NB, ch), lambda i: (i, 0))]
        out_shape = [jax.ShapeDtypeStruct((n, ch), jnp.float32)]
    return pl.pallas_call(
        _make_combine_body(ngroups, w is not None),
        grid=(n // NB,),
        in_specs=in_specs,
        out_specs=out_specs,
        out_shape=out_shape,
    )(*args)


def kernel(x, edge_index, W1, b1, W2, b2):
    src = edge_index[0].astype(jnp.int32)
    dst = edge_index[1].astype(jnp.int32)
    pad = EP - E
    srcp = jnp.concatenate([src, jnp.zeros((pad,), jnp.int32)]).reshape(EP // CH, CH)
    dstp = jnp.concatenate([dst, jnp.full((pad,), SINK, jnp.int32)]).reshape(EP // CH, CH)

    ones_h = jnp.ones((CH, 16), jnp.float32)
    zeros16 = jnp.zeros((NPAD, 16), jnp.float32)
    zeros32 = jnp.zeros((NPAD, 32), jnp.float32)

    p0, p1 = _deg_kernel(dstp, ones_h, zeros16)

    h1, *t = _mm_scale_split(x, W1, p0, p1, 4)
    a1 = _seg_sum(t, srcp, dstp, zeros32)

    outs = _combine(h1, p0, p1, b1, a1, w=W2, out_groups=2)
    h2, u0, u1 = outs
    a2 = _seg_sum([u0, u1], srcp, dstp, zeros32)

    (z,) = _combine(h2, p0, p1, b2, a2)
    return z
